# baseline (speedup 1.0000x reference)
import jax
import jax.numpy as jnp
from jax import lax
from jax.experimental import pallas as pl
from jax.experimental.pallas import tpu as pltpu
from jax.experimental.pallas import tpu_sc as plsc

BATCH = 16384
DIM = 128
NC = 2
NS = 16
NW = NC * NS
RPW = BATCH // NW
R = 128
C = RPW // R
LANES = 16


def _sep_bias_sc(x_hbm, lbl_hbm, scale_hbm, offset_hbm, o_hbm,
                 idx_v, s0, s1, b0, b1, x0, x1, x2,
                 sem_idx, sem_in0, sem_in1, sem_in2, sem_out0, sem_out1, sem_out2):
    wid = lax.axis_index("subcore") * NC + lax.axis_index("core")
    base = wid * RPW

    sbufs = (s0, s1)
    bbufs = (b0, b1)
    xbufs = (x0, x1, x2)
    sems_in = (sem_in0, sem_in1, sem_in2)
    sems_out = (sem_out0, sem_out1, sem_out2)

    cx0 = pltpu.async_copy(x_hbm.at[pl.ds(base, R)], xbufs[0], sems_in[0])
    idx_cp = pltpu.async_copy(lbl_hbm.at[pl.ds(base, RPW)], idx_v, sem_idx)

    def start_gathers(j, p2, p3):
        return (
            pltpu.async_copy(
                scale_hbm.at[idx_v.at[pl.ds(j * R, R)]], sbufs[p2], sems_in[p3]
            ),
            pltpu.async_copy(
                offset_hbm.at[idx_v.at[pl.ds(j * R, R)]], bbufs[p2], sems_in[p3]
            ),
        )

    def start_x(j, p3):
        return pltpu.async_copy(
            x_hbm.at[pl.ds(base + j * R, R)], xbufs[p3], sems_in[p3]
        )

    idx_cp.wait()
    pend = [None] * 3
    out_pend = [None] * 3
    pend[0] = (*start_gathers(0, 0, 0), cx0)
    for j in range(C):
        p2, p3 = j % 2, j % 3
        for d in pend[p3]:
            d.wait()
        if j + 1 < C:
            q3 = (j + 1) % 3
            if out_pend[q3] is not None:
                out_pend[q3].wait()
                out_pend[q3] = None
            pend[q3] = (*start_gathers(j + 1, (j + 1) % 2, q3), start_x(j + 1, q3))
        s_buf, b_buf, x_buf = sbufs[p2], bbufs[p2], xbufs[p3]

        @plsc.parallel_loop(0, R)
        def _(r):
            @plsc.parallel_loop(0, DIM, step=LANES)
            def _(c):
                rs, cs = pl.ds(r, 1), pl.ds(c, LANES)
                x_buf.at[rs, cs][...] = (
                    s_buf.at[rs, cs][...] * x_buf.at[rs, cs][...]
                    + b_buf.at[rs, cs][...]
                )

        out_pend[p3] = pltpu.async_copy(
            x_buf, o_hbm.at[pl.ds(base + j * R, R)], sems_out[p3]
        )
    for p in range(3):
        if out_pend[p] is not None:
            out_pend[p].wait()


def kernel(inputs, label, scale_table, offset_table):
    mesh = plsc.VectorSubcoreMesh(core_axis_name="core", subcore_axis_name="subcore")
    buf = pltpu.VMEM((R, DIM), jnp.float32)
    k = pl.kernel(
        _sep_bias_sc,
        out_type=jax.ShapeDtypeStruct((BATCH, DIM), jnp.float32),
        mesh=mesh,
        scratch_types=[
            pltpu.VMEM((RPW,), jnp.int32),
            buf, buf, buf, buf, buf, buf, buf,
            pltpu.SemaphoreType.DMA,
            pltpu.SemaphoreType.DMA,
            pltpu.SemaphoreType.DMA,
            pltpu.SemaphoreType.DMA,
            pltpu.SemaphoreType.DMA,
            pltpu.SemaphoreType.DMA,
            pltpu.SemaphoreType.DMA,
        ],
    )
    return k(inputs, label.astype(jnp.int32), scale_table, offset_table)


# speedup vs baseline: 1.1565x; 1.1565x over previous
import jax
import jax.numpy as jnp
from jax import lax
from jax.experimental import pallas as pl
from jax.experimental.pallas import tpu as pltpu
from jax.experimental.pallas import tpu_sc as plsc

BATCH = 16384
DIM = 128
NC = 2
NS = 16
NW = NC * NS
RPW = BATCH // NW
R = 128
C = RPW // R
LANES = 16


def _sep_bias_sc(x_hbm, lbl_hbm, scale_hbm, offset_hbm, o_hbm,
                 idx_v, s0, s1, b0, b1, x0, x1,
                 sem_idx, sem_in0, sem_in1, sem_out0, sem_out1):
    wid = lax.axis_index("subcore") * NC + lax.axis_index("core")
    base = wid * RPW

    sbufs = (s0, s1)
    bbufs = (b0, b1)
    xbufs = (x0, x1)
    sems_in = (sem_in0, sem_in1)
    sems_out = (sem_out0, sem_out1)

    def start_in(j, p):
        idx = idx_v.at[pl.ds(pl.multiple_of(j * R, R), R)]
        pltpu.async_copy(scale_hbm.at[idx], sbufs[p], sems_in[p])
        pltpu.async_copy(offset_hbm.at[idx], bbufs[p], sems_in[p])
        pltpu.async_copy(
            x_hbm.at[pl.ds(base + j * R, R)], xbufs[p], sems_in[p]
        )

    def wait_in(p):
        pltpu.make_async_copy(scale_hbm.at[pl.ds(0, R)], sbufs[p], sems_in[p]).wait()
        pltpu.make_async_copy(scale_hbm.at[pl.ds(0, R)], bbufs[p], sems_in[p]).wait()
        pltpu.make_async_copy(x_hbm.at[pl.ds(0, R)], xbufs[p], sems_in[p]).wait()

    def start_out(j, p):
        pltpu.async_copy(xbufs[p], o_hbm.at[pl.ds(base + j * R, R)], sems_out[p])

    def wait_out(p):
        pltpu.make_async_copy(xbufs[p], o_hbm.at[pl.ds(base, R)], sems_out[p]).wait()

    def compute(p):
        s_buf, b_buf, x_buf = sbufs[p], bbufs[p], xbufs[p]

        @plsc.parallel_loop(0, R)
        def _(r):
            for c in range(DIM // LANES):
                rs, cs = pl.ds(r, 1), pl.ds(c * LANES, LANES)
                x_buf.at[rs, cs][...] = (
                    s_buf.at[rs, cs][...] * x_buf.at[rs, cs][...]
                    + b_buf.at[rs, cs][...]
                )

    cx0 = pltpu.async_copy(x_hbm.at[pl.ds(base, R)], xbufs[0], sems_in[0])
    idx_cp = pltpu.async_copy(lbl_hbm.at[pl.ds(base, RPW)], idx_v, sem_idx)
    idx_cp.wait()
    idx = idx_v.at[pl.ds(0, R)]
    pltpu.async_copy(scale_hbm.at[idx], sbufs[0], sems_in[0])
    pltpu.async_copy(offset_hbm.at[idx], bbufs[0], sems_in[0])
    del cx0

    @pl.loop(0, C // 2)
    def _(t):
        j0 = pl.multiple_of(2 * t * R, R)
        wait_in(0)

        @pl.when(t > 0)
        def _():
            wait_out(1)

        start_in(2 * t + 1, 1)
        compute(0)
        start_out(2 * t, 0)
        wait_in(1)

        @pl.when(t < C // 2 - 1)
        def _():
            wait_out(0)
            start_in(2 * t + 2, 0)

        compute(1)
        start_out(2 * t + 1, 1)
        del j0

    wait_out(0)
    wait_out(1)


def kernel(inputs, label, scale_table, offset_table):
    mesh = plsc.VectorSubcoreMesh(core_axis_name="core", subcore_axis_name="subcore")
    buf = pltpu.VMEM((R, DIM), jnp.float32)
    k = pl.kernel(
        _sep_bias_sc,
        out_type=jax.ShapeDtypeStruct((BATCH, DIM), jnp.float32),
        mesh=mesh,
        scratch_types=[
            pltpu.VMEM((RPW,), jnp.int32),
            buf, buf, buf, buf, buf, buf,
            pltpu.SemaphoreType.DMA,
            pltpu.SemaphoreType.DMA,
            pltpu.SemaphoreType.DMA,
            pltpu.SemaphoreType.DMA,
            pltpu.SemaphoreType.DMA,
        ],
    )
    return k(inputs, label.astype(jnp.int32), scale_table, offset_table)
